# baseline (speedup 1.0000x reference)
import functools

import jax
import jax.numpy as jnp
from jax import lax
from jax.experimental import pallas as pl
from jax.experimental.pallas import tpu as pltpu
from jax.experimental.pallas import tpu_sc as plsc

N = 20000
M = 100
L = 16
NW = 32
NPAD = 20480
MPAD = 112
S = 7168
PER_W = S // NW
VECS = PER_W // L
TB = (NPAD - S) // 128
RB = TB // 8

FG_T = 0.5
BG_T = 0.3


def _roi_match_sc(planar, gtall):
    mesh = plsc.VectorSubcoreMesh(core_axis_name="c", subcore_axis_name="s")

    @functools.partial(
        pl.kernel,
        mesh=mesh,
        out_type=[
            jax.ShapeDtypeStruct((S,), jnp.float32),
            jax.ShapeDtypeStruct((S,), jnp.int32),
            jax.ShapeDtypeStruct((S,), jnp.int32),
        ],
        scratch_types=[
            pltpu.VMEM((4 * PER_W,), jnp.float32),
            pltpu.VMEM((5 * MPAD * L + MPAD,), jnp.float32),
            pltpu.VMEM((PER_W,), jnp.float32),
            pltpu.VMEM((PER_W,), jnp.int32),
            pltpu.VMEM((PER_W,), jnp.int32),
            pltpu.SemaphoreType.DMA,
            pltpu.SemaphoreType.DMA,
            pltpu.SemaphoreType.DMA,
        ],
    )
    def k(planar_hbm, gtall_hbm, vals_hbm, idxs_hbm, labs_hbm,
          prop_v, gt_v, vals_v, idxs_v, labs_v, sem1, sem2, sem3):
        wid = lax.axis_index("s") * 2 + lax.axis_index("c")
        base = wid * PER_W
        c1 = pltpu.async_copy(planar_hbm.at[wid], prop_v, sem1)
        c2 = pltpu.async_copy(gtall_hbm, gt_v, sem2)
        with jax.named_scope("dma_in"):
            c1.wait()
            c2.wait()

        def iou_one(p, parea, g, ga):
            ltx = jnp.maximum(g[0], p[0])
            lty = jnp.maximum(g[1], p[1])
            rbx = jnp.minimum(g[2], p[2])
            rby = jnp.minimum(g[3], p[3])
            w = jnp.maximum(rbx - ltx, 0.0)
            h = jnp.maximum(rby - lty, 0.0)
            inter = w * h
            return inter / (ga + parea - inter)

        def per_pair(v2, _):
            off_a = v2 * (2 * L)
            off_b = off_a + L
            pxa = [prop_v[pl.ds(c * PER_W + off_a, L)] for c in range(4)]
            pxb = [prop_v[pl.ds(c * PER_W + off_b, L)] for c in range(4)]
            parea_a = (pxa[2] - pxa[0]) * (pxa[3] - pxa[1])
            parea_b = (pxb[2] - pxb[0]) * (pxb[3] - pxb[1])

            def per_gt(m2, carry):
                va, ia, vb, ib = carry
                for u in range(2):
                    m = m2 * 2 + u
                    g = [gt_v[pl.ds(m * L + c * (MPAD * L), L)]
                         for c in range(4)]
                    ga = gt_v[pl.ds(m * L + 4 * (MPAD * L), L)]
                    iou_a = iou_one(pxa, parea_a, g, ga)
                    iou_b = iou_one(pxb, parea_b, g, ga)
                    ua = iou_a > va
                    ub = iou_b > vb
                    va = jnp.where(ua, iou_a, va)
                    ia = jnp.where(ua, m, ia)
                    vb = jnp.where(ub, iou_b, vb)
                    ib = jnp.where(ub, m, ib)
                return va, ia, vb, ib

            neg1 = jnp.full((L,), -1.0, jnp.float32)
            zero = jnp.zeros((L,), jnp.int32)
            va, ia, vb, ib = lax.fori_loop(
                0, M // 2, per_gt, (neg1, zero, neg1, zero))

            for off, bv, bi in ((off_a, va, ia), (off_b, vb, ib)):
                lbl_f = jnp.zeros((L,), jnp.float32)
                hi = lax.shift_right_logical(bi, 4)
                lo = jnp.bitwise_and(bi, 15)
                dnums = lax.GatherDimensionNumbers(
                    offset_dims=(), collapsed_slice_dims=(0,),
                    start_index_map=(0,))
                for j in range(MPAD // L):
                    row = gt_v[pl.ds(5 * MPAD * L + j * L, L)]
                    g = lax.gather(
                        row, lo[:, None], dnums, slice_sizes=(1,),
                        mode=lax.GatherScatterMode.PROMISE_IN_BOUNDS)
                    lbl_f = jnp.where(hi == j, g, lbl_f)
                below = bv < BG_T
                notfg = bv < FG_T
                idx_out = jnp.where(below, -1, jnp.where(notfg, -2, bi))
                lbl_out = jnp.where(
                    below, 0.0, jnp.where(notfg, -1.0, lbl_f)
                ).astype(jnp.int32)
                vals_v[pl.ds(off, L)] = bv
                idxs_v[pl.ds(off, L)] = idx_out
                labs_v[pl.ds(off, L)] = lbl_out
            return 0

        with jax.named_scope("mainloop"):
            lax.fori_loop(0, VECS // 2, per_pair, 0)

        o1 = pltpu.async_copy(vals_v, vals_hbm.at[pl.ds(base, PER_W)], sem1)
        o2 = pltpu.async_copy(idxs_v, idxs_hbm.at[pl.ds(base, PER_W)], sem2)
        o3 = pltpu.async_copy(labs_v, labs_hbm.at[pl.ds(base, PER_W)], sem3)
        o1.wait()
        o2.wait()
        o3.wait()

    return k(planar, gtall)


def _roi_match_tc(tprops, gt5, glab):
    def body(props_ref, gt_ref, glab_ref, vals_ref, idxs_ref, labs_ref):
        p = [props_ref[c] for c in range(4)]
        parea = (p[2] - p[0]) * (p[3] - p[1])

        def per_gt(m, carry):
            bv, bi, bl = carry
            ltx = jnp.maximum(gt_ref[0, m], p[0])
            lty = jnp.maximum(gt_ref[1, m], p[1])
            rbx = jnp.minimum(gt_ref[2, m], p[2])
            rby = jnp.minimum(gt_ref[3, m], p[3])
            w = jnp.maximum(rbx - ltx, 0.0)
            h = jnp.maximum(rby - lty, 0.0)
            inter = w * h
            iou = inter / (gt_ref[4, m] + parea - inter)
            upd = iou > bv
            bv = jnp.where(upd, iou, bv)
            bi = jnp.where(upd, m, bi)
            bl = jnp.where(upd, glab_ref[m], bl)
            return bv, bi, bl

        init = (jnp.full((TB, 128), -1.0, jnp.float32),
                jnp.zeros((TB, 128), jnp.int32),
                jnp.zeros((TB, 128), jnp.int32))
        bv, bi, bl = lax.fori_loop(0, M, per_gt, init)

        below = bv < BG_T
        notfg = bv < FG_T
        vals_ref[...] = bv
        idxs_ref[...] = jnp.where(below, -1, jnp.where(notfg, -2, bi))
        labs_ref[...] = jnp.where(below, 0, jnp.where(notfg, -1, bl))

    return pl.pallas_call(
        body,
        out_shape=[
            jax.ShapeDtypeStruct((TB, 128), jnp.float32),
            jax.ShapeDtypeStruct((TB, 128), jnp.int32),
            jax.ShapeDtypeStruct((TB, 128), jnp.int32),
        ],
        in_specs=[
            pl.BlockSpec(memory_space=pltpu.VMEM),
            pl.BlockSpec(memory_space=pltpu.SMEM),
            pl.BlockSpec(memory_space=pltpu.SMEM),
        ],
    )(tprops, gt5, glab)


def kernel(proposal_boxes, gt_boxes, gt_labels):
    p = jnp.pad(proposal_boxes.astype(jnp.float32), ((0, NPAD - N), (0, 0)))
    planar = p.T
    g = jnp.pad(gt_boxes.astype(jnp.float32), ((0, MPAD - M), (0, 0)))
    gt = g.T
    garea = (gt[2] - gt[0]) * (gt[3] - gt[1])
    gt5 = jnp.concatenate([gt, garea[None]], axis=0)
    gtb = jnp.broadcast_to(gt5[:, :, None], (5, MPAD, L))
    glab = jnp.pad(gt_labels.astype(jnp.int32), (0, MPAD - M))

    tprops = planar[:, S:].reshape(4, TB, 128)
    sprops = (planar[:, :S].reshape(4, NW, PER_W).transpose(1, 0, 2)
              .reshape(NW, 4 * PER_W))
    gtall = jnp.concatenate([gtb.reshape(-1), glab.astype(jnp.float32)])
    sc_vals, sc_idxs, sc_labs = _roi_match_sc(sprops, gtall)
    tc_vals, tc_idxs, tc_labs = _roi_match_tc(tprops, gt5, glab)

    vals = jnp.concatenate([sc_vals, tc_vals.reshape(-1)[:N - S]])
    idxs = jnp.concatenate([sc_idxs, tc_idxs.reshape(-1)[:N - S]])
    labs = jnp.concatenate([sc_labs, tc_labs.reshape(-1)[:N - S]])
    return vals, idxs, labs


# speedup vs baseline: 1.0052x; 1.0052x over previous
import functools

import jax
import jax.numpy as jnp
from jax import lax
from jax.experimental import pallas as pl
from jax.experimental.pallas import tpu as pltpu
from jax.experimental.pallas import tpu_sc as plsc

N = 20000
M = 100
L = 16
NW = 32
NPAD = 20480
MPAD = 112
S = 7168
PER_W = S // NW
VECS = PER_W // L
TB = (NPAD - S) // 128
RB = TB // 8

FG_T = 0.5
BG_T = 0.3


def _roi_match_sc(planar, gtall):
    mesh = plsc.VectorSubcoreMesh(core_axis_name="c", subcore_axis_name="s")

    @functools.partial(
        pl.kernel,
        mesh=mesh,
        out_type=[
            jax.ShapeDtypeStruct((S,), jnp.float32),
            jax.ShapeDtypeStruct((S,), jnp.int32),
            jax.ShapeDtypeStruct((S,), jnp.int32),
        ],
        scratch_types=[
            pltpu.VMEM((4 * PER_W,), jnp.float32),
            pltpu.VMEM((5 * MPAD * L,), jnp.float32),
            pltpu.VMEM((6 * MPAD,), jnp.float32),
            pltpu.VMEM((PER_W,), jnp.float32),
            pltpu.VMEM((PER_W,), jnp.int32),
            pltpu.VMEM((PER_W,), jnp.int32),
            pltpu.SemaphoreType.DMA,
            pltpu.SemaphoreType.DMA,
            pltpu.SemaphoreType.DMA,
        ],
    )
    def k(planar_hbm, gtall_hbm, vals_hbm, idxs_hbm, labs_hbm,
          prop_v, gt_v, gtc_v, vals_v, idxs_v, labs_v, sem1, sem2, sem3):
        wid = lax.axis_index("s") * 2 + lax.axis_index("c")
        base = wid * PER_W
        c1 = pltpu.async_copy(planar_hbm.at[wid], prop_v, sem1)
        c2 = pltpu.async_copy(gtall_hbm, gtc_v, sem2)
        with jax.named_scope("dma_in"):
            c1.wait()
            c2.wait()

        bdnums = lax.GatherDimensionNumbers(
            offset_dims=(), collapsed_slice_dims=(0,), start_index_map=(0,))
        with jax.named_scope("bcast"):
            for c in range(5):
                for grp in range(MPAD // L):
                    v = gtc_v[pl.ds(c * MPAD + grp * L, L)]
                    for i in range(L):
                        idx = jnp.full((L, 1), i, jnp.int32)
                        row = lax.gather(
                            v, idx, bdnums, slice_sizes=(1,),
                            mode=lax.GatherScatterMode.PROMISE_IN_BOUNDS)
                        gt_v[pl.ds(c * (MPAD * L) + (grp * L + i) * L, L)] = row

        def iou_one(p, parea, g, ga):
            ltx = jnp.maximum(g[0], p[0])
            lty = jnp.maximum(g[1], p[1])
            rbx = jnp.minimum(g[2], p[2])
            rby = jnp.minimum(g[3], p[3])
            w = jnp.maximum(rbx - ltx, 0.0)
            h = jnp.maximum(rby - lty, 0.0)
            inter = w * h
            return inter / (ga + parea - inter)

        def per_pair(v2, _):
            off_a = v2 * (2 * L)
            off_b = off_a + L
            pxa = [prop_v[pl.ds(c * PER_W + off_a, L)] for c in range(4)]
            pxb = [prop_v[pl.ds(c * PER_W + off_b, L)] for c in range(4)]
            parea_a = (pxa[2] - pxa[0]) * (pxa[3] - pxa[1])
            parea_b = (pxb[2] - pxb[0]) * (pxb[3] - pxb[1])

            def per_gt(m2, carry):
                va, ia, vb, ib = carry
                for u in range(2):
                    m = m2 * 2 + u
                    g = [gt_v[pl.ds(m * L + c * (MPAD * L), L)]
                         for c in range(4)]
                    ga = gt_v[pl.ds(m * L + 4 * (MPAD * L), L)]
                    iou_a = iou_one(pxa, parea_a, g, ga)
                    iou_b = iou_one(pxb, parea_b, g, ga)
                    ua = iou_a > va
                    ub = iou_b > vb
                    va = jnp.where(ua, iou_a, va)
                    ia = jnp.where(ua, m, ia)
                    vb = jnp.where(ub, iou_b, vb)
                    ib = jnp.where(ub, m, ib)
                return va, ia, vb, ib

            neg1 = jnp.full((L,), -1.0, jnp.float32)
            zero = jnp.zeros((L,), jnp.int32)
            va, ia, vb, ib = lax.fori_loop(
                0, M // 2, per_gt, (neg1, zero, neg1, zero))

            for off, bv, bi in ((off_a, va, ia), (off_b, vb, ib)):
                lbl_f = jnp.zeros((L,), jnp.float32)
                hi = lax.shift_right_logical(bi, 4)
                lo = jnp.bitwise_and(bi, 15)
                dnums = lax.GatherDimensionNumbers(
                    offset_dims=(), collapsed_slice_dims=(0,),
                    start_index_map=(0,))
                for j in range(MPAD // L):
                    row = gtc_v[pl.ds(5 * MPAD + j * L, L)]
                    g = lax.gather(
                        row, lo[:, None], dnums, slice_sizes=(1,),
                        mode=lax.GatherScatterMode.PROMISE_IN_BOUNDS)
                    lbl_f = jnp.where(hi == j, g, lbl_f)
                below = bv < BG_T
                notfg = bv < FG_T
                idx_out = jnp.where(below, -1, jnp.where(notfg, -2, bi))
                lbl_out = jnp.where(
                    below, 0.0, jnp.where(notfg, -1.0, lbl_f)
                ).astype(jnp.int32)
                vals_v[pl.ds(off, L)] = bv
                idxs_v[pl.ds(off, L)] = idx_out
                labs_v[pl.ds(off, L)] = lbl_out
            return 0

        with jax.named_scope("mainloop"):
            lax.fori_loop(0, VECS // 2, per_pair, 0)

        o1 = pltpu.async_copy(vals_v, vals_hbm.at[pl.ds(base, PER_W)], sem1)
        o2 = pltpu.async_copy(idxs_v, idxs_hbm.at[pl.ds(base, PER_W)], sem2)
        o3 = pltpu.async_copy(labs_v, labs_hbm.at[pl.ds(base, PER_W)], sem3)
        o1.wait()
        o2.wait()
        o3.wait()

    return k(planar, gtall)


def _roi_match_tc(tprops, gt5, glab):
    def body(props_ref, gt_ref, glab_ref, vals_ref, idxs_ref, labs_ref):
        p = [props_ref[c] for c in range(4)]
        parea = (p[2] - p[0]) * (p[3] - p[1])

        def per_gt(m, carry):
            bv, bi, bl = carry
            ltx = jnp.maximum(gt_ref[0, m], p[0])
            lty = jnp.maximum(gt_ref[1, m], p[1])
            rbx = jnp.minimum(gt_ref[2, m], p[2])
            rby = jnp.minimum(gt_ref[3, m], p[3])
            w = jnp.maximum(rbx - ltx, 0.0)
            h = jnp.maximum(rby - lty, 0.0)
            inter = w * h
            iou = inter / (gt_ref[4, m] + parea - inter)
            upd = iou > bv
            bv = jnp.where(upd, iou, bv)
            bi = jnp.where(upd, m, bi)
            bl = jnp.where(upd, glab_ref[m], bl)
            return bv, bi, bl

        init = (jnp.full((TB, 128), -1.0, jnp.float32),
                jnp.zeros((TB, 128), jnp.int32),
                jnp.zeros((TB, 128), jnp.int32))
        bv, bi, bl = lax.fori_loop(0, M, per_gt, init)

        below = bv < BG_T
        notfg = bv < FG_T
        vals_ref[...] = bv
        idxs_ref[...] = jnp.where(below, -1, jnp.where(notfg, -2, bi))
        labs_ref[...] = jnp.where(below, 0, jnp.where(notfg, -1, bl))

    return pl.pallas_call(
        body,
        out_shape=[
            jax.ShapeDtypeStruct((TB, 128), jnp.float32),
            jax.ShapeDtypeStruct((TB, 128), jnp.int32),
            jax.ShapeDtypeStruct((TB, 128), jnp.int32),
        ],
        in_specs=[
            pl.BlockSpec(memory_space=pltpu.VMEM),
            pl.BlockSpec(memory_space=pltpu.SMEM),
            pl.BlockSpec(memory_space=pltpu.SMEM),
        ],
    )(tprops, gt5, glab)


def kernel(proposal_boxes, gt_boxes, gt_labels):
    p = jnp.pad(proposal_boxes.astype(jnp.float32), ((0, NPAD - N), (0, 0)))
    planar = p.T
    g = jnp.pad(gt_boxes.astype(jnp.float32), ((0, MPAD - M), (0, 0)))
    gt = g.T
    garea = (gt[2] - gt[0]) * (gt[3] - gt[1])
    gt5 = jnp.concatenate([gt, garea[None]], axis=0)
    glab = jnp.pad(gt_labels.astype(jnp.int32), (0, MPAD - M))

    tprops = planar[:, S:].reshape(4, TB, 128)
    sprops = (planar[:, :S].reshape(4, NW, PER_W).transpose(1, 0, 2)
              .reshape(NW, 4 * PER_W))
    gtall = jnp.concatenate([gt5.reshape(-1), glab.astype(jnp.float32)])
    sc_vals, sc_idxs, sc_labs = _roi_match_sc(sprops, gtall)
    tc_vals, tc_idxs, tc_labs = _roi_match_tc(tprops, gt5, glab)

    vals = jnp.concatenate([sc_vals, tc_vals.reshape(-1)[:N - S]])
    idxs = jnp.concatenate([sc_idxs, tc_idxs.reshape(-1)[:N - S]])
    labs = jnp.concatenate([sc_labs, tc_labs.reshape(-1)[:N - S]])
    return vals, idxs, labs


# speedup vs baseline: 1.0102x; 1.0049x over previous
import functools

import jax
import jax.numpy as jnp
from jax import lax
from jax.experimental import pallas as pl
from jax.experimental.pallas import tpu as pltpu
from jax.experimental.pallas import tpu_sc as plsc

N = 20000
M = 100
L = 16
NW = 32
NPAD = 20480
MPAD = 112
S = 7168
PER_W = S // NW
VECS = PER_W // L
TB = (NPAD - S) // 128
RB = TB // 8

FG_T = 0.5
BG_T = 0.3


def _roi_match_sc(planar, gtall):
    mesh = plsc.VectorSubcoreMesh(core_axis_name="c", subcore_axis_name="s")

    @functools.partial(
        pl.kernel,
        mesh=mesh,
        out_type=[
            jax.ShapeDtypeStruct((S,), jnp.float32),
            jax.ShapeDtypeStruct((S,), jnp.int32),
            jax.ShapeDtypeStruct((S,), jnp.int32),
        ],
        scratch_types=[
            pltpu.VMEM((4 * PER_W,), jnp.float32),
            pltpu.VMEM((5 * MPAD * L,), jnp.float32),
            pltpu.VMEM((6 * MPAD,), jnp.float32),
            pltpu.VMEM((PER_W,), jnp.float32),
            pltpu.VMEM((PER_W,), jnp.int32),
            pltpu.VMEM((PER_W,), jnp.int32),
            pltpu.SemaphoreType.DMA,
            pltpu.SemaphoreType.DMA,
            pltpu.SemaphoreType.DMA,
        ],
    )
    def k(planar_hbm, gtall_hbm, vals_hbm, idxs_hbm, labs_hbm,
          prop_v, gt_v, gtc_v, vals_v, idxs_v, labs_v, sem1, sem2, sem3):
        wid = lax.axis_index("s") * 2 + lax.axis_index("c")
        base = wid * PER_W
        c1 = pltpu.async_copy(planar_hbm.at[wid], prop_v, sem1)
        c2 = pltpu.async_copy(gtall_hbm, gtc_v, sem2)
        c1.wait()
        c2.wait()

        bdnums = lax.GatherDimensionNumbers(
            offset_dims=(), collapsed_slice_dims=(0,), start_index_map=(0,))
        for c in range(5):
            for grp in range(MPAD // L):
                v = gtc_v[pl.ds(c * MPAD + grp * L, L)]
                for i in range(L):
                    idx = jnp.full((L, 1), i, jnp.int32)
                    row = lax.gather(
                        v, idx, bdnums, slice_sizes=(1,),
                        mode=lax.GatherScatterMode.PROMISE_IN_BOUNDS)
                    gt_v[pl.ds(c * (MPAD * L) + (grp * L + i) * L, L)] = row

        def iou_one(p, parea, g, ga):
            ltx = jnp.maximum(g[0], p[0])
            lty = jnp.maximum(g[1], p[1])
            rbx = jnp.minimum(g[2], p[2])
            rby = jnp.minimum(g[3], p[3])
            w = jnp.maximum(rbx - ltx, 0.0)
            h = jnp.maximum(rby - lty, 0.0)
            inter = w * h
            return inter / (ga + parea - inter)

        def per_pair(v2, _):
            off_a = v2 * (2 * L)
            off_b = off_a + L
            pxa = [prop_v[pl.ds(c * PER_W + off_a, L)] for c in range(4)]
            pxb = [prop_v[pl.ds(c * PER_W + off_b, L)] for c in range(4)]
            parea_a = (pxa[2] - pxa[0]) * (pxa[3] - pxa[1])
            parea_b = (pxb[2] - pxb[0]) * (pxb[3] - pxb[1])

            def per_gt(m2, carry):
                va, ia, vb, ib = carry
                for u in range(2):
                    m = m2 * 2 + u
                    g = [gt_v[pl.ds(m * L + c * (MPAD * L), L)]
                         for c in range(4)]
                    ga = gt_v[pl.ds(m * L + 4 * (MPAD * L), L)]
                    iou_a = iou_one(pxa, parea_a, g, ga)
                    iou_b = iou_one(pxb, parea_b, g, ga)
                    ua = iou_a > va
                    ub = iou_b > vb
                    va = jnp.where(ua, iou_a, va)
                    ia = jnp.where(ua, m, ia)
                    vb = jnp.where(ub, iou_b, vb)
                    ib = jnp.where(ub, m, ib)
                return va, ia, vb, ib

            neg1 = jnp.full((L,), -1.0, jnp.float32)
            zero = jnp.zeros((L,), jnp.int32)
            va, ia, vb, ib = lax.fori_loop(
                0, M // 2, per_gt, (neg1, zero, neg1, zero))

            for off, bv, bi in ((off_a, va, ia), (off_b, vb, ib)):
                lbl_f = jnp.zeros((L,), jnp.float32)
                hi = lax.shift_right_logical(bi, 4)
                lo = jnp.bitwise_and(bi, 15)
                dnums = lax.GatherDimensionNumbers(
                    offset_dims=(), collapsed_slice_dims=(0,),
                    start_index_map=(0,))
                for j in range(MPAD // L):
                    row = gtc_v[pl.ds(5 * MPAD + j * L, L)]
                    g = lax.gather(
                        row, lo[:, None], dnums, slice_sizes=(1,),
                        mode=lax.GatherScatterMode.PROMISE_IN_BOUNDS)
                    lbl_f = jnp.where(hi == j, g, lbl_f)
                below = bv < BG_T
                notfg = bv < FG_T
                idx_out = jnp.where(below, -1, jnp.where(notfg, -2, bi))
                lbl_out = jnp.where(
                    below, 0.0, jnp.where(notfg, -1.0, lbl_f)
                ).astype(jnp.int32)
                vals_v[pl.ds(off, L)] = bv
                idxs_v[pl.ds(off, L)] = idx_out
                labs_v[pl.ds(off, L)] = lbl_out
            return 0

        lax.fori_loop(0, VECS // 2, per_pair, 0)

        o1 = pltpu.async_copy(vals_v, vals_hbm.at[pl.ds(base, PER_W)], sem1)
        o2 = pltpu.async_copy(idxs_v, idxs_hbm.at[pl.ds(base, PER_W)], sem2)
        o3 = pltpu.async_copy(labs_v, labs_hbm.at[pl.ds(base, PER_W)], sem3)
        o1.wait()
        o2.wait()
        o3.wait()

    return k(planar, gtall)


def _roi_match_tc(tprops, gt5, glab):
    def body(props_ref, gt_ref, glab_ref, vals_ref, idxs_ref, labs_ref):
        p = [props_ref[c] for c in range(4)]
        parea = (p[2] - p[0]) * (p[3] - p[1])

        def per_gt(m, carry):
            bv, bi, bl = carry
            ltx = jnp.maximum(gt_ref[0, m], p[0])
            lty = jnp.maximum(gt_ref[1, m], p[1])
            rbx = jnp.minimum(gt_ref[2, m], p[2])
            rby = jnp.minimum(gt_ref[3, m], p[3])
            w = jnp.maximum(rbx - ltx, 0.0)
            h = jnp.maximum(rby - lty, 0.0)
            inter = w * h
            iou = inter / (gt_ref[4, m] + parea - inter)
            upd = iou > bv
            bv = jnp.where(upd, iou, bv)
            bi = jnp.where(upd, m, bi)
            bl = jnp.where(upd, glab_ref[m], bl)
            return bv, bi, bl

        init = (jnp.full((TB, 128), -1.0, jnp.float32),
                jnp.zeros((TB, 128), jnp.int32),
                jnp.zeros((TB, 128), jnp.int32))
        bv, bi, bl = lax.fori_loop(0, M, per_gt, init)

        below = bv < BG_T
        notfg = bv < FG_T
        vals_ref[...] = bv
        idxs_ref[...] = jnp.where(below, -1, jnp.where(notfg, -2, bi))
        labs_ref[...] = jnp.where(below, 0, jnp.where(notfg, -1, bl))

    return pl.pallas_call(
        body,
        out_shape=[
            jax.ShapeDtypeStruct((TB, 128), jnp.float32),
            jax.ShapeDtypeStruct((TB, 128), jnp.int32),
            jax.ShapeDtypeStruct((TB, 128), jnp.int32),
        ],
        in_specs=[
            pl.BlockSpec(memory_space=pltpu.VMEM),
            pl.BlockSpec(memory_space=pltpu.SMEM),
            pl.BlockSpec(memory_space=pltpu.SMEM),
        ],
    )(tprops, gt5, glab)


def kernel(proposal_boxes, gt_boxes, gt_labels):
    p = jnp.pad(proposal_boxes.astype(jnp.float32), ((0, NPAD - N), (0, 0)))
    planar = p.T
    g = jnp.pad(gt_boxes.astype(jnp.float32), ((0, MPAD - M), (0, 0)))
    gt = g.T
    garea = (gt[2] - gt[0]) * (gt[3] - gt[1])
    gt5 = jnp.concatenate([gt, garea[None]], axis=0)
    glab = jnp.pad(gt_labels.astype(jnp.int32), (0, MPAD - M))

    tprops = planar[:, S:].reshape(4, TB, 128)
    sprops = (planar[:, :S].reshape(4, NW, PER_W).transpose(1, 0, 2)
              .reshape(NW, 4 * PER_W))
    gtall = jnp.concatenate([gt5.reshape(-1), glab.astype(jnp.float32)])
    sc_vals, sc_idxs, sc_labs = _roi_match_sc(sprops, gtall)
    tc_vals, tc_idxs, tc_labs = _roi_match_tc(tprops, gt5, glab)

    vals = jnp.concatenate([sc_vals, tc_vals.reshape(-1)[:N - S]])
    idxs = jnp.concatenate([sc_idxs, tc_idxs.reshape(-1)[:N - S]])
    labs = jnp.concatenate([sc_labs, tc_labs.reshape(-1)[:N - S]])
    return vals, idxs, labs
